# Initial kernel scaffold; baseline (speedup 1.0000x reference)
import jax
import jax.numpy as jnp
from jax.experimental import pallas as pl


def kernel(inputs, embedding):
    raise NotImplementedError("write your pallas kernel here")


# speedup vs baseline: 1.3081x; 1.3081x over previous
import functools

import jax
import jax.numpy as jnp
from jax import lax
from jax.experimental import pallas as pl
from jax.experimental.pallas import tpu as pltpu
from jax.experimental.pallas import tpu_sc as plsc

_N = 8192
_K = 8192
_D = 256
_TN = 256
_TK = 1024
_COMMITMENT_COST = 0.25
_INT_MAX = 2**31 - 1


def _argmin_body(x_ref, rn_ref, emb_ref, idx_ref, loss_ref):
    x = x_ref[...]
    rn = rn_ref[...]
    best_val = None
    best_idx = None
    for kc in range(_K // _TK):
        e = emb_ref[pl.ds(kc * _TK, _TK), :]
        m = lax.dot_general(e, x, (((1,), (1,)), ((), ())),
                            preferred_element_type=jnp.float32)
        cn = jnp.sum(e * e, axis=1, keepdims=True)
        dist = (rn - 2.0 * m) + cn
        cmin = jnp.min(dist, axis=0, keepdims=True)
        ids = lax.broadcasted_iota(jnp.int32, (_TK, _TN), 0) + (kc * _TK)
        marg = jnp.min(jnp.where(dist == cmin, ids, _INT_MAX),
                       axis=0, keepdims=True)
        if kc == 0:
            best_val, best_idx = cmin, marg
        else:
            take = cmin < best_val
            best_idx = jnp.where(take, marg, best_idx)
            best_val = jnp.where(take, cmin, best_val)
    idx_ref[0] = best_idx
    partial = jnp.sum(best_val, axis=1, keepdims=True) * (
        _COMMITMENT_COST / (_N * _D))

    @pl.when(pl.program_id(0) == 0)
    def _init():
        loss_ref[...] = partial

    @pl.when(pl.program_id(0) > 0)
    def _acc():
        loss_ref[...] = loss_ref[...] + partial


def _argmin_call(flat, rn_row, embedding, interpret=False):
    return pl.pallas_call(
        _argmin_body,
        grid=(_N // _TN,),
        in_specs=[
            pl.BlockSpec((_TN, _D), lambda i: (i, 0)),
            pl.BlockSpec((1, _TN), lambda i: (0, i)),
            pl.BlockSpec((_K, _D), lambda i: (0, 0)),
        ],
        out_specs=[
            pl.BlockSpec((1, 1, _TN), lambda i: (i, 0, 0)),
            pl.BlockSpec((1, 1), lambda i: (0, 0)),
        ],
        out_shape=[
            jax.ShapeDtypeStruct((_N // _TN, 1, _TN), jnp.int32),
            jax.ShapeDtypeStruct((1, 1), jnp.float32),
        ],
        interpret=interpret,
    )(flat, rn_row, embedding)


def _make_sc_gather_hist():
    info = plsc.get_sparse_core_info()
    NC, NS = info.num_cores, info.num_subcores
    NW = NC * NS
    BPW = _N // NW
    CH = 128
    NCH = BPW // CH
    mesh = plsc.VectorSubcoreMesh(core_axis_name="c", subcore_axis_name="s")

    @functools.partial(
        pl.kernel,
        out_type=[
            jax.ShapeDtypeStruct((_N, _D), jnp.float32),
            jax.ShapeDtypeStruct((NC, _K), jnp.float32),
        ],
        mesh=mesh,
        scratch_types=[
            pltpu.VMEM((NCH, CH), jnp.int32),
            pltpu.VMEM((BPW, _D), jnp.float32),
            pltpu.VMEM((CH,), jnp.float32),
            pltpu.VMEM((_K // NS,), jnp.float32),
            pltpu.VMEM_SHARED((_K,), jnp.float32),
            pltpu.SemaphoreType.DMA,
        ],
    )
    def sc_kernel(idx_hbm, emb_hbm, quant_hbm, counts_hbm,
                  idx_v, rows_v, ones_v, zeros_v, hist_sh, sem):
        cid = lax.axis_index("c")
        sid = lax.axis_index("s")
        wid = sid * NC + cid
        base = wid * BPW

        pltpu.sync_copy(idx_hbm.at[pl.ds(wid * NCH, NCH)], idx_v)

        def _fill_ones(i, _):
            ones_v[pl.ds(i * 16, 16)] = jnp.ones((16,), jnp.float32)
            return 0
        lax.fori_loop(0, CH // 16, _fill_ones, 0)

        def _fill_zeros(i, _):
            zeros_v[pl.ds(i * 16, 16)] = jnp.zeros((16,), jnp.float32)
            return 0
        lax.fori_loop(0, (_K // NS) // 16, _fill_zeros, 0)

        pltpu.sync_copy(zeros_v, hist_sh.at[pl.ds(sid * (_K // NS), _K // NS)])
        plsc.subcore_barrier()

        for ch in range(NCH):
            pltpu.async_copy(emb_hbm.at[idx_v.at[ch]],
                             rows_v.at[pl.ds(ch * CH, CH)], sem).wait()
            pltpu.sync_copy(ones_v, hist_sh.at[idx_v.at[ch]], add=True)

        pltpu.sync_copy(rows_v, quant_hbm.at[pl.ds(base, BPW)])

        plsc.subcore_barrier()

        @pl.when(sid == 0)
        def _():
            pltpu.sync_copy(hist_sh, counts_hbm.at[cid])

    return sc_kernel, NC, NCH, CH


def _finalize_body(parts_ref, counts_ref, perp_ref):
    p = parts_ref[...]
    c = jnp.sum(p, axis=0, keepdims=True)
    counts_ref[...] = c
    probs = c * (1.0 / _N)
    ent = jnp.sum(probs * jnp.log(probs + 1e-10), axis=1, keepdims=True)
    perp_ref[...] = jnp.exp(-ent)


def _finalize_call(parts, interpret=False):
    nc = parts.shape[0]
    return pl.pallas_call(
        _finalize_body,
        in_specs=[pl.BlockSpec((nc, _K), lambda: (0, 0))],
        out_specs=[
            pl.BlockSpec((1, _K), lambda: (0, 0)),
            pl.BlockSpec((1, 1), lambda: (0, 0)),
        ],
        out_shape=[
            jax.ShapeDtypeStruct((1, _K), jnp.float32),
            jax.ShapeDtypeStruct((1, 1), jnp.float32),
        ],
        interpret=interpret,
    )(parts)


def kernel(inputs, embedding):
    flat = inputs.reshape(_N, _D)
    rn_row = jnp.sum(flat * flat, axis=1, keepdims=True).reshape(1, _N)
    idx3, loss = _argmin_call(flat, rn_row, embedding)
    idx = idx3.reshape(_N)

    sc_kernel, nc, nch, ch = _make_sc_gather_hist()
    idx_chunked = idx.reshape(_N // ch, ch)
    quantized, parts = sc_kernel(idx_chunked, embedding)

    counts, perp = _finalize_call(parts)

    quantized_st = quantized.reshape(inputs.shape)
    return (loss.reshape(()), quantized_st, counts.reshape(_K),
            embedding, perp.reshape(()))
